# Initial kernel scaffold; baseline (speedup 1.0000x reference)
import jax
import jax.numpy as jnp
from jax.experimental import pallas as pl


def kernel(rec_node_attr, rec_edge_attr, rec_edge_index, rec_xyz, lig_node_attr, lig_edge_attr, lig_edge_index, labelidx, rec_Win, rec_Wmsg, rec_Wself, rec_Wout, lig_Win, lig_Wmsg, lig_Wself, lig_Wout, Wr, br, Wl, bl, Wgr, bgr, Wgl, bgl):
    raise NotImplementedError("write your pallas kernel here")


# speedup vs baseline: 9.6672x; 9.6672x over previous
import functools

import jax
import jax.numpy as jnp
from jax import lax
from jax.experimental import pallas as pl
from jax.experimental.pallas import tpu as pltpu
from jax.experimental.pallas import tpu_sc as plsc

_NC = 2
_NS = 16
_CH = 128
_BLK = 2048

_F32 = jnp.float32


def _cdiv(a, b):
    return (a + b - 1) // b


def _sc_segment_sum(R, NSB, SBC, DHH):
    ZR = R // _NS
    mesh = plsc.VectorSubcoreMesh(core_axis_name="c", subcore_axis_name="s",
                                  num_cores=_NC, num_subcores=_NS)
    out_type = [jax.ShapeDtypeStruct((_NC, R, DHH), _F32),
                jax.ShapeDtypeStruct((R,), _F32)]
    scratch = [
        pltpu.VMEM((2, SBC, _CH), jnp.int32),
        pltpu.VMEM((2, SBC, _CH), jnp.int32),
        pltpu.VMEM((2, SBC, _CH), _F32),
        pltpu.VMEM((2, _CH, DHH), _F32),
        pltpu.VMEM_SHARED((R, DHH), _F32),
        pltpu.VMEM_SHARED((R,), _F32),
        pltpu.SemaphoreType.DMA,
        pltpu.SemaphoreType.DMA,
        pltpu.SemaphoreType.DMA,
        pltpu.SemaphoreType.DMA,
        pltpu.SemaphoreType.DMA,
        pltpu.SemaphoreType.DMA,
    ]

    def body(hw2, srcs, dsts, zrows, ea, zs, agg_out, s_out,
             src_v, dst_v, ea_v, rows_v, agg_sp, s_sp,
             sg0, sg1, si0, si1, se0, se1):
        sg = (sg0, sg1)
        si = (si0, si1)
        se = (se0, se1)
        cid = lax.axis_index("c")
        sid = lax.axis_index("s")
        base = sid * ZR
        is0 = cid == 0
        off = cid * R

        pltpu.sync_copy(zrows, agg_sp.at[pl.ds(base, ZR)])

        @pl.when(is0)
        def _():
            pltpu.sync_copy(zs, s_sp.at[pl.ds(base, ZR)])

        plsc.subcore_barrier()

        def sb_issue(sb, b):
            pltpu.async_copy(srcs.at[sid, sb], src_v.at[b], si[b])
            pltpu.async_copy(dsts.at[sid, sb], dst_v.at[b], si[b])

            @pl.when(is0)
            def _():
                pltpu.async_copy(ea.at[sid, sb], ea_v.at[b], se[b])

        def sb_wait(sb, b):
            pltpu.make_async_copy(srcs.at[sid, sb], src_v.at[b],
                                  si[b]).wait()
            pltpu.make_async_copy(dsts.at[sid, sb], dst_v.at[b],
                                  si[b]).wait()

            @pl.when(is0)
            def _():
                pltpu.make_async_copy(ea.at[sid, sb], ea_v.at[b],
                                      se[b]).wait()

        def adjust(b, j):
            for k in range(_CH // 16):
                sl = pl.ds(k * 16, 16)
                src_v[b, j, sl] = src_v[b, j, sl] + off

        def g_issue(b, j, r):
            pltpu.async_copy(hw2.at[src_v.at[b, j]], rows_v.at[r], sg[r])

        def g_wait(b, j, r):
            pltpu.make_async_copy(hw2.at[src_v.at[b, j]], rows_v.at[r],
                                  sg[r]).wait()

        sb_issue(0, 0)
        for sb in range(NSB):
            b = sb % 2
            sb_wait(sb, b)
            if sb + 1 < NSB:
                sb_issue(sb + 1, 1 - b)
            adjust(b, 0)
            g_issue(b, 0, 0)

            @pl.loop(0, SBC, step=2)
            def _(jc, b=b):
                for r in range(2):
                    jj = jc + r

                    def _nxt(b=b, jj=jj, r=r):
                        adjust(b, jj + 1)
                        g_issue(b, jj + 1, 1 - r)

                    pl.when(jj + 1 < SBC)(_nxt)
                    g_wait(b, jj, r)
                    pltpu.sync_copy(rows_v.at[r],
                                    agg_sp.at[dst_v.at[b, jj]], add=True)

                    def _sadd(b=b, jj=jj):
                        pltpu.sync_copy(ea_v.at[b, jj],
                                        s_sp.at[dst_v.at[b, jj]], add=True)

                    pl.when(is0)(_sadd)

        plsc.subcore_barrier()
        pltpu.sync_copy(agg_sp.at[pl.ds(base, ZR)],
                        agg_out.at[cid, pl.ds(base, ZR)])

        @pl.when(is0)
        def _():
            pltpu.sync_copy(s_sp.at[pl.ds(base, ZR)],
                            s_out.at[pl.ds(base, ZR)])

    return pl.kernel(body, out_type=out_type, mesh=mesh,
                     scratch_types=scratch,
                     compiler_params=pltpu.CompilerParams(
                         use_tc_tiling_on_sc=False))


def _mm(a, b):
    return jnp.dot(a, b, preferred_element_type=_F32)


def _split_half(h):
    dh = h.shape[-1]
    return jnp.stack([h[:, :dh // 2], h[:, dh // 2:]], axis=0)


def _rec_in(x, win, w1):
    R, din = x.shape
    dh = win.shape[1]

    def body(x_r, win_r, w1_r, h0_r, hw_r):
        h = jnp.maximum(_mm(x_r[...], win_r[...]), 0.0)
        h0_r[...] = h
        hw_r[...] = _split_half(_mm(h, w1_r[...]))

    return pl.pallas_call(
        body,
        grid=(R // _BLK,),
        in_specs=[pl.BlockSpec((_BLK, din), lambda i: (i, 0)),
                  pl.BlockSpec((din, dh), lambda i: (0, 0)),
                  pl.BlockSpec((dh, dh), lambda i: (0, 0))],
        out_specs=[pl.BlockSpec((_BLK, dh), lambda i: (i, 0)),
                   pl.BlockSpec((2, _BLK, dh // 2), lambda i: (0, i, 0))],
        out_shape=[jax.ShapeDtypeStruct((R, dh), _F32),
                   jax.ShapeDtypeStruct((2, R, dh // 2), _F32)],
    )(x, win, w1)


def _rec_mid(aggp, sv, h0, wself, w2, w1n):
    _, R, dhh = aggp.shape
    dh = 2 * dhh

    def body(ap_r, sv_r, h0_r, ws_r, w2_r, w1_r, h1_r, hw_r):
        agg = jnp.concatenate([ap_r[0], ap_r[1]], axis=-1)
        h1 = jnp.maximum(
            agg + sv_r[...] * w2_r[...] + _mm(h0_r[...], ws_r[...]), 0.0)
        h1_r[...] = h1
        hw_r[...] = _split_half(_mm(h1, w1_r[...]))

    return pl.pallas_call(
        body,
        grid=(R // _BLK,),
        in_specs=[pl.BlockSpec((_NC, _BLK, dhh), lambda i: (0, i, 0)),
                  pl.BlockSpec((_BLK, 1), lambda i: (i, 0)),
                  pl.BlockSpec((_BLK, dh), lambda i: (i, 0)),
                  pl.BlockSpec((dh, dh), lambda i: (0, 0)),
                  pl.BlockSpec((1, dh), lambda i: (0, 0)),
                  pl.BlockSpec((dh, dh), lambda i: (0, 0))],
        out_specs=[pl.BlockSpec((_BLK, dh), lambda i: (i, 0)),
                   pl.BlockSpec((2, _BLK, dhh), lambda i: (0, i, 0))],
        out_shape=[jax.ShapeDtypeStruct((R, dh), _F32),
                   jax.ShapeDtypeStruct((2, R, dhh), _F32)],
    )(aggp, sv, h0, wself, w2, w1n)


def _rec_logits(aggp, ss, h1, wself, w2, wrow, c, n_valid):
    _, R, dhh = aggp.shape
    dh = 2 * dhh

    def body(ap_r, sp_r, h1_r, ws_r, w2_r, wr_r, c_r, lm_r):
        i = pl.program_id(0)
        agg = jnp.concatenate([ap_r[0], ap_r[1]], axis=-1)
        h2 = jnp.maximum(
            agg + sp_r[...] * w2_r[...] + _mm(h1_r[...], ws_r[...]), 0.0)
        logits = jnp.sum(h2 * wr_r[...], axis=1, keepdims=True) + c_r[0, 0]
        valid = (i * _BLK +
                 lax.broadcasted_iota(jnp.int32, (_BLK, 1), 0)) < n_valid
        lm_r[...] = jnp.where(valid, logits, -1e30)

    return pl.pallas_call(
        body,
        grid=(R // _BLK,),
        in_specs=[pl.BlockSpec((_NC, _BLK, dhh), lambda i: (0, i, 0)),
                  pl.BlockSpec((_BLK, 1), lambda i: (i, 0)),
                  pl.BlockSpec((_BLK, dh), lambda i: (i, 0)),
                  pl.BlockSpec((dh, dh), lambda i: (0, 0)),
                  pl.BlockSpec((1, dh), lambda i: (0, 0)),
                  pl.BlockSpec((1, dh), lambda i: (0, 0)),
                  pl.BlockSpec((1, 1), lambda i: (0, 0))],
        out_specs=pl.BlockSpec((_BLK, 1), lambda i: (i, 0)),
        out_shape=jax.ShapeDtypeStruct((R, 1), _F32),
    )(aggp, ss, h1, wself, w2, wrow, c)


def _softmax_y(lm2, x0, x1, x2):
    rows, lanes = lm2.shape

    def body(lm_r, x0_r, x1_r, x2_r, a_r, y_r):
        lm = lm_r[...]
        m = jnp.max(lm)
        e = jnp.exp(lm - m)
        a = e / jnp.sum(e)
        a_r[...] = a
        ys = [jnp.sum(a * x_r[...]).reshape(1, 1)
              for x_r in (x0_r, x1_r, x2_r)]
        y_r[...] = jnp.concatenate(ys, axis=1)

    return pl.pallas_call(
        body,
        out_shape=[jax.ShapeDtypeStruct((rows, lanes), _F32),
                   jax.ShapeDtypeStruct((1, 3), _F32)],
    )(lm2, x0, x1, x2)


def _lig_head(x, ea, src_col, dst_row, labelidx, win, wm_top, wm_bot, wself,
              wout, wl, bl_row, wr_t, rwout_t, br_row):
    nl, din = x.shape
    el = ea.shape[0]
    nlayer, dh, _ = wself.shape

    def body(x_r, ea_r, src_r, dst_r, lab_r, win_r, wmt_r, wmb_r, ws_r,
             wout_r, wl_r, bl_r, wrt_r, rwt_r, br_r, wrow_r, c_r):
        h = jnp.maximum(_mm(x_r[...], win_r[...]), 0.0)
        src_oh = (src_r[...] ==
                  lax.broadcasted_iota(jnp.int32, (el, nl), 1)).astype(_F32)
        dst_oht = (lax.broadcasted_iota(jnp.int32, (nl, el), 0) ==
                   dst_r[...]).astype(_F32)
        for l in range(nlayer):
            m = _mm(_mm(src_oh, h), wmt_r[l]) + _mm(ea_r[...], wmb_r[l])
            agg = _mm(dst_oht, m)
            h = jnp.maximum(agg + _mm(h, ws_r[l]), 0.0)
        hs = _mm(h, wout_r[...])
        h_l = _mm(lab_r[...], hs)
        h_l_w = _mm(h_l, wl_r[...]) + bl_r[...]
        wrow_r[...] = _mm(_mm(h_l_w, wrt_r[...]), rwt_r[...])
        c_r[...] = jnp.sum(h_l_w * br_r[...], axis=1, keepdims=True)

    return pl.pallas_call(
        body,
        out_shape=[jax.ShapeDtypeStruct((1, dh), _F32),
                   jax.ShapeDtypeStruct((1, 1), _F32)],
    )(x, ea, src_col, dst_row, labelidx, win, wm_top, wm_bot, wself, wout,
      wl, bl_row, wr_t, rwout_t, br_row)


def kernel(rec_node_attr, rec_edge_attr, rec_edge_index, rec_xyz,
           lig_node_attr, lig_edge_attr, lig_edge_index, labelidx,
           rec_Win, rec_Wmsg, rec_Wself, rec_Wout,
           lig_Win, lig_Wmsg, lig_Wself, lig_Wout,
           Wr, br, Wl, bl, Wgr, bgr, Wgl, bgl):
    N = rec_node_attr.shape[0]
    E = rec_edge_index.shape[1]
    dh = rec_Win.shape[1]
    dhh = dh // 2

    R = _cdiv(N + 1, _BLK) * _BLK
    nch = _cdiv(E, _NS * _CH)
    NSB = _cdiv(nch, 50)
    SBC = _cdiv(nch, NSB)
    SBC = SBC + (SBC % 2)
    EPAD = _NS * NSB * SBC * _CH
    ZR = R // _NS

    src = rec_edge_index[0].astype(jnp.int32)
    dst = rec_edge_index[1].astype(jnp.int32)
    src_p = jnp.pad(src, (0, EPAD - E)).reshape(_NS, NSB, SBC, _CH)
    dst_p = jnp.pad(dst, (0, EPAD - E),
                    constant_values=N).reshape(_NS, NSB, SBC, _CH)
    ea_p = jnp.pad(rec_edge_attr[:, 0], (0, EPAD - E)).reshape(
        _NS, NSB, SBC, _CH)
    x_p = jnp.pad(rec_node_attr, ((0, R - N), (0, 0)))
    xyz_p = jnp.pad(rec_xyz, ((0, R - N), (0, 0)))
    zrows = jnp.zeros((ZR, dhh), _F32)
    zs = jnp.zeros((ZR,), _F32)
    w1 = [rec_Wmsg[l][:dh] for l in range(rec_Wmsg.shape[0])]
    w2 = [rec_Wmsg[l][dh:] for l in range(rec_Wmsg.shape[0])]
    lig_wm_top = lig_Wmsg[:, :dh]
    lig_wm_bot = lig_Wmsg[:, dh:]

    wrow, c = _lig_head(
        lig_node_attr, lig_edge_attr,
        lig_edge_index[0].astype(jnp.int32)[:, None],
        lig_edge_index[1].astype(jnp.int32)[None, :],
        labelidx, lig_Win, lig_wm_top, lig_wm_bot, lig_Wself, lig_Wout,
        Wl, bl[None, :], Wr.T, rec_Wout.T, br[None, :])

    h0, hw0 = _rec_in(x_p, rec_Win, w1[0])
    sc_call = _sc_segment_sum(R, NSB, SBC, dhh)
    agg0, s_flat = sc_call(hw0.reshape(2 * R, dhh), src_p, dst_p, zrows,
                           ea_p, zs)
    s_vec = s_flat.reshape(R, 1)
    h1, hw1 = _rec_mid(agg0, s_vec, h0, rec_Wself[0], w2[0], w1[1])
    agg1, _ = sc_call(hw1.reshape(2 * R, dhh), src_p, dst_p, zrows,
                      ea_p, zs)
    lm = _rec_logits(agg1, s_vec, h1, rec_Wself[1], w2[1], wrow, c, N)
    xcols = [xyz_p[:, k].reshape(R // 128, 128) for k in range(3)]
    a2d, yrec = _softmax_y(lm.reshape(R // 128, 128), *xcols)

    A = a2d.reshape(R)[:N][None, :]
    return (yrec.reshape(1, 1, 3), A)
